# baseline (speedup 1.0000x reference)
import math

import jax
import jax.numpy as jnp
import numpy as np
from jax.experimental import pallas as pl
from jax.experimental.pallas import tpu as pltpu

B, M, T = 2, 4, 512
D_IN = 512
D_MODEL = 512
E, TOPK = 8, 2
H_EXP = 1024
TP = 128
GRU_H = 128
AK = 64
AV = 64
N_CLS = 10
N = B * M * T
SEQ = B * M
NT = M * T
NBB = NT // T
NEG = -1e30


def _pos_encoding_np(t, d):
    position = np.arange(t)[:, None].astype(np.float32)
    div = np.exp(np.arange(0, d, 2).astype(np.float32) * (-math.log(10000.0) / d))
    pe = np.zeros((t, d), dtype=np.float32)
    pe[:, 0::2] = np.sin(position * div)
    pe[:, 1::2] = np.cos(position * div)
    return pe


def _dot(a, b):
    return jnp.dot(a, b, preferred_element_type=jnp.float32)


def _mega_kernel(x_ref, pe_ref, u_ref, rt_ref, st_ref,
                 w_in_ref, b_in_ref, w_tok_ref, b_tok_ref, w_q_ref, b_q_ref,
                 wk_t_ref, w_v_ref, b_v_ref, wih_t_ref, b_ihx_ref,
                 whh_t_ref, b_hhn_ref,
                 wg1a_ref, wg1b_ref, b_g1_ref, w_g2_ref, b_g2_ref,
                 h16_ref, hsum_ref, gates_ref, gs_ref,
                 gi_s, hs_s, proc_s):
    c_u = wih_t_ref[0:1, :]
    c_rs = _dot(w_v_ref[...], wih_t_ref[1:, :])
    c_0 = _dot(b_v_ref[...], wih_t_ref[1:, :]) + b_ihx_ref[...]
    col = jax.lax.broadcasted_iota(jnp.int32, (T, M), 1)
    hsums = []
    for s in range(SEQ):
        h = (_dot(x_ref[s * T:(s + 1) * T, :], w_in_ref[...])
             + b_in_ref[...] + pe_ref[...])
        h16_ref[s * T:(s + 1) * T, :] = h.astype(jnp.bfloat16)
        hsums.append(jnp.sum(h, axis=0, keepdims=True))
        proc = jnp.maximum(_dot(h, w_tok_ref[...]) + b_tok_ref[...], 0.0)
        proc_s[s] = proc
        q = _dot(proc, w_q_ref[...]) + b_q_ref[...]
        qk = _dot(q, wk_t_ref[...]) * (1.0 / math.sqrt(AK))
        rb = rt_ref[s]
        sb = st_ref[s]
        scores = rb * qk[:, 0:1] + sb * qk[:, 1:2]
        scores = jnp.where(col == (s % M), NEG, scores)
        w = jnp.exp(scores - jnp.max(scores, axis=1, keepdims=True))
        w = w / jnp.sum(w, axis=1, keepdims=True)
        wr = jnp.sum(w * rb, axis=1, keepdims=True)
        ws = jnp.sum(w * sb, axis=1, keepdims=True)
        gi_s[s] = (u_ref[s] * c_u + wr * c_rs[0:1, :] + ws * c_rs[1:2, :] + c_0)
    hsum_ref[0:1, :] = hsums[0] + hsums[1] + hsums[2] + hsums[3]
    hsum_ref[1:2, :] = hsums[4] + hsums[5] + hsums[6] + hsums[7]

    def step(t, h):
        gi = gi_s[:, t, :]
        gh = _dot(h, whh_t_ref[...])
        rz = jnp.tanh((gi[:, :2 * GRU_H] + gh[:, :2 * GRU_H]) * 0.5) * 0.5 + 0.5
        r = rz[:, :GRU_H]
        z = rz[:, GRU_H:]
        n = jnp.tanh(gi[:, 2 * GRU_H:] + r * (gh[:, 2 * GRU_H:] + b_hhn_ref[...]))
        h_new = n + z * (h - n)
        hs_s[:, t, :] = h_new
        return h_new

    def step4(j, h):
        t = j * 4
        h = step(t, h)
        h = step(t + 1, h)
        h = step(t + 2, h)
        return step(t + 3, h)

    jax.lax.fori_loop(0, T // 4, step4, jnp.zeros((SEQ, GRU_H), jnp.float32))

    pv = proc_s[...].reshape(N, TP)
    gv = hs_s[...].reshape(N, GRU_H)
    hid = jnp.maximum(_dot(pv, wg1a_ref[...]) + _dot(gv, wg1b_ref[...])
                      + b_g1_ref[...], 0.0)
    logits = _dot(hid, w_g2_ref[...]) + b_g2_ref[...]
    idx = jax.lax.broadcasted_iota(jnp.int32, (N, E), 1)
    v1 = jnp.max(logits, axis=1, keepdims=True)
    i1 = jnp.min(jnp.where(logits == v1, idx, E), axis=1, keepdims=True)
    masked = jnp.where(idx == i1, NEG, logits)
    v2 = jnp.max(masked, axis=1, keepdims=True)
    i2 = jnp.min(jnp.where(masked == v2, idx, E), axis=1, keepdims=True)
    e2 = jnp.exp(v2 - v1)
    g1 = 1.0 / (1.0 + e2)
    gates = jnp.where(idx == i1, g1, 0.0) + jnp.where(idx == i2, e2 * g1, 0.0)
    gates_ref[...] = gates
    g3 = gates.reshape(SEQ, T, E)
    gs_ref[0:1, :] = jnp.sum(jnp.sum(g3[:M], axis=0), axis=0, keepdims=True)
    gs_ref[1:2, :] = jnp.sum(jnp.sum(g3[M:], axis=0), axis=0, keepdims=True)


def _expert_final_kernel(h16_ref, w1_ref, b1_ref, g_ref,
                         hsum_ref, gs_ref, w2_ref, b2_ref, wout_ref, bout_ref,
                         out_ref, s_acc):
    e = pl.program_id(0)
    b = pl.program_id(1)
    k = pl.program_id(2)
    eh = jnp.maximum(_dot(h16_ref[...], w1_ref[0]) + b1_ref[0], 0.0)
    contrib = _dot(g_ref[0], eh)
    row = b * E + e

    @pl.when(k == 0)
    def _():
        s_acc[pl.ds(row, 1), :] = contrib

    @pl.when(k != 0)
    def _():
        s_acc[pl.ds(row, 1), :] += contrib

    @pl.when((e == E - 1) & (b == B - 1) & (k == NBB - 1))
    def _():
        s2 = s_acc[...].reshape(B, E * H_EXP).astype(jnp.bfloat16)
        ymoe = (_dot(s2, w2_ref[...]) + _dot(gs_ref[...], b2_ref[...])) * (1.0 / NT)
        y = hsum_ref[...] * (1.0 / NT) + ymoe
        out_ref[...] = _dot(y, wout_ref[...]) + bout_ref[...]


def kernel(x, U, R, S, W_in, b_in, W_tok, b_tok, W_q, b_q, W_k, b_k, W_v, b_v,
           W_ih, b_ih, W_hh, b_hh, W_g1, b_g1, W_g2, b_g2, W_e1, b_e1,
           W_e2, b_e2, W_out, b_out):
    f32 = jnp.float32
    bf16 = jnp.bfloat16
    pe = jnp.asarray(_pos_encoding_np(T, D_MODEL))
    x2 = x.reshape(N, D_IN)
    u3 = U.reshape(SEQ, T, 1)
    rt = jnp.transpose(R.reshape(SEQ, M, T), (0, 2, 1))
    st = jnp.transpose(S.reshape(SEQ, M, T), (0, 2, 1))
    b_ihx = b_ih + jnp.concatenate([b_hh[:2 * GRU_H], jnp.zeros((GRU_H,), f32)])

    def row2(v):
        return v.reshape(1, -1)

    full = lambda *shape: pl.BlockSpec(shape, lambda: tuple(0 for _ in shape))
    h16, hsum, gates, gs = pl.pallas_call(
        _mega_kernel,
        in_specs=[full(N, D_IN), full(T, D_MODEL), full(SEQ, T, 1),
                  full(SEQ, T, M), full(SEQ, T, M),
                  full(D_IN, D_MODEL), full(1, D_MODEL),
                  full(D_MODEL, TP), full(1, TP),
                  full(TP, AK), full(1, AK),
                  full(AK, 2), full(2, AV), full(1, AV),
                  full(1 + AV, 3 * GRU_H), full(1, 3 * GRU_H),
                  full(GRU_H, 3 * GRU_H), full(1, GRU_H),
                  full(TP, (TP + GRU_H) // 2), full(GRU_H, (TP + GRU_H) // 2),
                  full(1, (TP + GRU_H) // 2),
                  full((TP + GRU_H) // 2, E), full(1, E)],
        out_specs=[full(N, D_MODEL), full(B, D_MODEL), full(N, E), full(B, E)],
        out_shape=[
            jax.ShapeDtypeStruct((N, D_MODEL), bf16),
            jax.ShapeDtypeStruct((B, D_MODEL), f32),
            jax.ShapeDtypeStruct((N, E), f32),
            jax.ShapeDtypeStruct((B, E), f32),
        ],
        scratch_shapes=[
            pltpu.VMEM((SEQ, T, 3 * GRU_H), f32),
            pltpu.VMEM((SEQ, T, GRU_H), f32),
            pltpu.VMEM((SEQ, T, TP), f32),
        ],
    )(x2, pe, u3, rt, st, W_in, row2(b_in), W_tok, row2(b_tok),
      W_q, row2(b_q), W_k.T, W_v, row2(b_v), W_ih.T, row2(b_ihx),
      W_hh.T, row2(b_hh[2 * GRU_H:]),
      W_g1[:TP], W_g1[TP:], row2(b_g1), W_g2, row2(b_g2))

    gates_t = gates.T.reshape(E, 1, N)
    out = pl.pallas_call(
        _expert_final_kernel,
        grid=(E, B, NBB),
        in_specs=[
            pl.BlockSpec((T, D_MODEL), lambda e, b, k: (b * NBB + k, 0)),
            pl.BlockSpec((1, D_MODEL, H_EXP), lambda e, b, k: (e, 0, 0)),
            pl.BlockSpec((1, 1, H_EXP), lambda e, b, k: (e, 0, 0)),
            pl.BlockSpec((1, 1, T), lambda e, b, k: (e, 0, b * NBB + k)),
            pl.BlockSpec((B, D_MODEL), lambda e, b, k: (0, 0)),
            pl.BlockSpec((B, E), lambda e, b, k: (0, 0)),
            pl.BlockSpec((E * H_EXP, D_MODEL), lambda e, b, k: (0, 0)),
            pl.BlockSpec((E, D_MODEL), lambda e, b, k: (0, 0)),
            pl.BlockSpec((D_MODEL, N_CLS), lambda e, b, k: (0, 0)),
            pl.BlockSpec((1, N_CLS), lambda e, b, k: (0, 0)),
        ],
        out_specs=pl.BlockSpec((B, N_CLS), lambda e, b, k: (0, 0)),
        out_shape=jax.ShapeDtypeStruct((B, N_CLS), f32),
        scratch_shapes=[pltpu.VMEM((B * E, H_EXP), f32)],
    )(h16, W_e1.astype(bf16), b_e1.reshape(E, 1, H_EXP), gates_t,
      hsum, gs, W_e2.reshape(E * H_EXP, D_MODEL).astype(bf16), b_e2,
      W_out, row2(b_out))
    return out


# speedup vs baseline: 1.1325x; 1.1325x over previous
import math

import jax
import jax.numpy as jnp
import numpy as np
from jax.experimental import pallas as pl
from jax.experimental.pallas import tpu as pltpu

B, M, T = 2, 4, 512
D_IN = 512
D_MODEL = 512
E, TOPK = 8, 2
H_EXP = 1024
TP = 128
GRU_H = 128
AK = 64
AV = 64
N_CLS = 10
N = B * M * T
SEQ = B * M
NT = M * T
NBB = NT // T
NEG = -1e30


def _pos_encoding_np(t, d):
    position = np.arange(t)[:, None].astype(np.float32)
    div = np.exp(np.arange(0, d, 2).astype(np.float32) * (-math.log(10000.0) / d))
    pe = np.zeros((t, d), dtype=np.float32)
    pe[:, 0::2] = np.sin(position * div)
    pe[:, 1::2] = np.cos(position * div)
    return pe


def _dot(a, b):
    return jnp.dot(a, b, preferred_element_type=jnp.float32)


def _mega_kernel(x_ref, pe_ref, u_ref, rt_ref, st_ref,
                 w_in_ref, b_in_ref, w_tok_ref, b_tok_ref, w_q_ref, b_q_ref,
                 wk_t_ref, w_v_ref, b_v_ref, wih_t_ref, b_ihx_ref,
                 whh_t_ref, b_hhn_ref,
                 wg1a_ref, wg1b_ref, b_g1_ref, w_g2_ref, b_g2_ref,
                 h16_ref, hsum_ref, gates_ref, gs_ref,
                 gi_s, hs_s, proc_s):
    c_u = wih_t_ref[0:1, :]
    c_rs = _dot(w_v_ref[...], wih_t_ref[1:, :])
    c_0 = _dot(b_v_ref[...], wih_t_ref[1:, :]) + b_ihx_ref[...]
    col = jax.lax.broadcasted_iota(jnp.int32, (T, M), 1)
    hsums = []
    for s in range(SEQ):
        h = (_dot(x_ref[s * T:(s + 1) * T, :], w_in_ref[...])
             + b_in_ref[...] + pe_ref[...])
        h16_ref[s * T:(s + 1) * T, :] = h.astype(jnp.bfloat16)
        hsums.append(jnp.sum(h, axis=0, keepdims=True))
        proc = jnp.maximum(_dot(h, w_tok_ref[...]) + b_tok_ref[...], 0.0)
        proc_s[s] = proc
        q = _dot(proc, w_q_ref[...]) + b_q_ref[...]
        qk = _dot(q, wk_t_ref[...]) * (1.0 / math.sqrt(AK))
        rb = rt_ref[s]
        sb = st_ref[s]
        scores = rb * qk[:, 0:1] + sb * qk[:, 1:2]
        scores = jnp.where(col == (s % M), NEG, scores)
        w = jnp.exp(scores - jnp.max(scores, axis=1, keepdims=True))
        w = w / jnp.sum(w, axis=1, keepdims=True)
        wr = jnp.sum(w * rb, axis=1, keepdims=True)
        ws = jnp.sum(w * sb, axis=1, keepdims=True)
        gi_s[s] = (u_ref[s] * c_u + wr * c_rs[0:1, :] + ws * c_rs[1:2, :] + c_0)
    hsum_ref[0:1, :] = hsums[0] + hsums[1] + hsums[2] + hsums[3]
    hsum_ref[1:2, :] = hsums[4] + hsums[5] + hsums[6] + hsums[7]

    def step(t, h):
        gi = gi_s[:, t, :]
        gh = _dot(h, whh_t_ref[...])
        rz = jnp.tanh((gi[:, :2 * GRU_H] + gh[:, :2 * GRU_H]) * 0.5) * 0.5 + 0.5
        r = rz[:, :GRU_H]
        z = rz[:, GRU_H:]
        n = jnp.tanh(gi[:, 2 * GRU_H:] + r * (gh[:, 2 * GRU_H:] + b_hhn_ref[...]))
        h_new = n + z * (h - n)
        hs_s[:, t, :] = h_new
        return h_new

    def step4(j, h):
        t = j * 4
        h = step(t, h)
        h = step(t + 1, h)
        h = step(t + 2, h)
        return step(t + 3, h)

    jax.lax.fori_loop(0, T // 4, step4, jnp.zeros((SEQ, GRU_H), jnp.float32))

    pv = proc_s[...].reshape(N, TP)
    gv = hs_s[...].reshape(N, GRU_H)
    hid = jnp.maximum(_dot(pv, wg1a_ref[...]) + _dot(gv, wg1b_ref[...])
                      + b_g1_ref[...], 0.0)
    logits = _dot(hid, w_g2_ref[...]) + b_g2_ref[...]
    idx = jax.lax.broadcasted_iota(jnp.int32, (N, E), 1)
    v1 = jnp.max(logits, axis=1, keepdims=True)
    i1 = jnp.min(jnp.where(logits == v1, idx, E), axis=1, keepdims=True)
    masked = jnp.where(idx == i1, NEG, logits)
    v2 = jnp.max(masked, axis=1, keepdims=True)
    i2 = jnp.min(jnp.where(masked == v2, idx, E), axis=1, keepdims=True)
    e2 = jnp.exp(v2 - v1)
    g1 = 1.0 / (1.0 + e2)
    gates = jnp.where(idx == i1, g1, 0.0) + jnp.where(idx == i2, e2 * g1, 0.0)
    gates_ref[...] = gates
    g3 = gates.reshape(SEQ, T, E)
    gs_ref[0:1, :] = jnp.sum(jnp.sum(g3[:M], axis=0), axis=0, keepdims=True)
    gs_ref[1:2, :] = jnp.sum(jnp.sum(g3[M:], axis=0), axis=0, keepdims=True)


def _expert_final_kernel(h16_ref, w1_ref, g_ref,
                         b1_ref, hsum_ref, gs_ref, w2_ref, b2_ref,
                         wout_ref, bout_ref, out_ref, s_acc, w1s):
    e = pl.program_id(0)
    b = pl.program_id(1)
    k = pl.program_id(2)
    blk = b * NBB + k

    @pl.when((b == 0) & (k == 0))
    def _():
        w1s[...] = w1_ref[0].astype(jnp.bfloat16)

    h16 = h16_ref[pl.ds(blk * T, T), :]
    eh = jnp.maximum(_dot(h16, w1s[...]) + b1_ref[0], 0.0)
    g = g_ref[pl.ds(e, 1), 0, pl.ds(blk * T, T)]
    contrib = _dot(g, eh)
    row = b * E + e

    @pl.when(k == 0)
    def _():
        s_acc[pl.ds(row, 1), :] = contrib

    @pl.when(k != 0)
    def _():
        s_acc[pl.ds(row, 1), :] += contrib

    @pl.when((e == E - 1) & (b == B - 1) & (k == NBB - 1))
    def _():
        s2 = s_acc[...].reshape(B, E * H_EXP)
        ymoe = (_dot(s2, w2_ref[...]) + _dot(gs_ref[...], b2_ref[...])) * (1.0 / NT)
        y = hsum_ref[...] * (1.0 / NT) + ymoe
        out_ref[...] = _dot(y, wout_ref[...]) + bout_ref[...]


def kernel(x, U, R, S, W_in, b_in, W_tok, b_tok, W_q, b_q, W_k, b_k, W_v, b_v,
           W_ih, b_ih, W_hh, b_hh, W_g1, b_g1, W_g2, b_g2, W_e1, b_e1,
           W_e2, b_e2, W_out, b_out):
    f32 = jnp.float32
    bf16 = jnp.bfloat16
    pe = jnp.asarray(_pos_encoding_np(T, D_MODEL))
    x2 = x.reshape(N, D_IN)
    u3 = U.reshape(SEQ, T, 1)
    rt = jnp.transpose(R.reshape(SEQ, M, T), (0, 2, 1))
    st = jnp.transpose(S.reshape(SEQ, M, T), (0, 2, 1))
    b_ihx = b_ih + jnp.concatenate([b_hh[:2 * GRU_H], jnp.zeros((GRU_H,), f32)])

    def row2(v):
        return v.reshape(1, -1)

    full = lambda *shape: pl.BlockSpec(shape, lambda: tuple(0 for _ in shape))
    h16, hsum, gates, gs = pl.pallas_call(
        _mega_kernel,
        in_specs=[full(N, D_IN), full(T, D_MODEL), full(SEQ, T, 1),
                  full(SEQ, T, M), full(SEQ, T, M),
                  full(D_IN, D_MODEL), full(1, D_MODEL),
                  full(D_MODEL, TP), full(1, TP),
                  full(TP, AK), full(1, AK),
                  full(AK, 2), full(2, AV), full(1, AV),
                  full(1 + AV, 3 * GRU_H), full(1, 3 * GRU_H),
                  full(GRU_H, 3 * GRU_H), full(1, GRU_H),
                  full(TP, (TP + GRU_H) // 2), full(GRU_H, (TP + GRU_H) // 2),
                  full(1, (TP + GRU_H) // 2),
                  full((TP + GRU_H) // 2, E), full(1, E)],
        out_specs=[full(N, D_MODEL), full(B, D_MODEL), full(N, E), full(B, E)],
        out_shape=[
            jax.ShapeDtypeStruct((N, D_MODEL), bf16),
            jax.ShapeDtypeStruct((B, D_MODEL), f32),
            jax.ShapeDtypeStruct((N, E), f32),
            jax.ShapeDtypeStruct((B, E), f32),
        ],
        scratch_shapes=[
            pltpu.VMEM((SEQ, T, 3 * GRU_H), f32),
            pltpu.VMEM((SEQ, T, GRU_H), f32),
            pltpu.VMEM((SEQ, T, TP), f32),
        ],
    )(x2, pe, u3, rt, st, W_in, row2(b_in), W_tok, row2(b_tok),
      W_q, row2(b_q), W_k.T, W_v, row2(b_v), W_ih.T, row2(b_ihx),
      W_hh.T, row2(b_hh[2 * GRU_H:]),
      W_g1[:TP], W_g1[TP:], row2(b_g1), W_g2, row2(b_g2))

    gates_t = gates.T.reshape(E, 1, N)
    out = pl.pallas_call(
        _expert_final_kernel,
        grid=(E, B, NBB),
        in_specs=[
            pl.BlockSpec((N, D_MODEL), lambda e, b, k: (0, 0)),
            pl.BlockSpec((1, D_MODEL, H_EXP), lambda e, b, k: (e, 0, 0)),
            pl.BlockSpec((E, 1, N), lambda e, b, k: (0, 0, 0)),
            pl.BlockSpec((1, 1, H_EXP), lambda e, b, k: (e, 0, 0)),
            pl.BlockSpec((B, D_MODEL), lambda e, b, k: (0, 0)),
            pl.BlockSpec((B, E), lambda e, b, k: (0, 0)),
            pl.BlockSpec((E * H_EXP, D_MODEL), lambda e, b, k: (0, 0)),
            pl.BlockSpec((E, D_MODEL), lambda e, b, k: (0, 0)),
            pl.BlockSpec((D_MODEL, N_CLS), lambda e, b, k: (0, 0)),
            pl.BlockSpec((1, N_CLS), lambda e, b, k: (0, 0)),
        ],
        out_specs=pl.BlockSpec((B, N_CLS), lambda e, b, k: (0, 0)),
        out_shape=jax.ShapeDtypeStruct((B, N_CLS), f32),
        scratch_shapes=[pltpu.VMEM((B * E, H_EXP), f32),
                        pltpu.VMEM((D_MODEL, H_EXP), bf16)],
    )(h16, W_e1, gates_t, b_e1.reshape(E, 1, H_EXP),
      hsum, gs, W_e2.reshape(E * H_EXP, D_MODEL), b_e2,
      W_out, row2(b_out))
    return out
